# Initial kernel scaffold; baseline (speedup 1.0000x reference)
import jax
import jax.numpy as jnp
from jax.experimental import pallas as pl


def kernel(hidden_features, batch, current_epoch, sw1, sb1, sw2, sb2, sw3, sb3, fw1, fb1, fw2, fb2):
    raise NotImplementedError("write your pallas kernel here")


# speedup vs baseline: 4.2712x; 4.2712x over previous
import jax
import jax.numpy as jnp
import numpy as np
from jax.experimental import pallas as pl
from jax.experimental.pallas import tpu as pltpu

N = 10000
D_HID = 128
EMB = 8
K = 16
NPAD = 10240
BM = 128
GRID = NPAD // BM
INV_R2 = -1.0 / (0.3 ** 2)


def _topk_body(s_blk, sT, sq_row, sqb, idx_ref, key_ref, w_ref):
    i = pl.program_id(0)
    dot = jax.lax.dot_general(
        s_blk[...], sT[...], (((1,), (0,)), ((), ())),
        preferred_element_type=jnp.float32)
    dist = (sqb[...][:, 0:1] + sq_row[...][0:1, :]) - 2.0 * dot
    lane = jax.lax.broadcasted_iota(jnp.int32, (BM, NPAD), 1)
    dist = jnp.where(lane >= N, jnp.float32(np.inf), dist)
    rows = jax.lax.broadcasted_iota(jnp.int32, (BM, 1), 0) + i * BM
    big = jnp.int32(2 ** 30)
    for k in range(K):
        m = jnp.min(dist, axis=1, keepdims=True)
        amin = jnp.min(jnp.where(dist == m, lane, big), axis=1,
                       keepdims=True)
        idx_ref[:, k:k + 1] = amin
        key_ref[:, k:k + 1] = amin * N + rows
        w_ref[:, k:k + 1] = jnp.exp((-1.0 * m) / 0.09)
        dist = jnp.where(lane == amin, jnp.float32(np.inf), dist)


def _topk(s_pad, sq_pad):
    sT = s_pad.T
    sq_row = jnp.tile(sq_pad[None, :], (8, 1))
    sqb = jnp.tile(sq_pad[:, None], (1, 8))
    out_shapes = (
        jax.ShapeDtypeStruct((NPAD, K), jnp.int32),
        jax.ShapeDtypeStruct((NPAD, K), jnp.int32),
        jax.ShapeDtypeStruct((NPAD, K), jnp.float32),
    )
    blk = lambda i: (i, 0)
    return pl.pallas_call(
        _topk_body,
        grid=(GRID,),
        in_specs=[
            pl.BlockSpec((BM, EMB), blk),
            pl.BlockSpec((EMB, NPAD), lambda i: (0, 0)),
            pl.BlockSpec((8, NPAD), lambda i: (0, 0)),
            pl.BlockSpec((BM, 8), blk),
        ],
        out_specs=(
            pl.BlockSpec((BM, K), blk),
            pl.BlockSpec((BM, K), blk),
            pl.BlockSpec((BM, K), blk),
        ),
        out_shape=out_shapes,
    )(s_pad, sT, sq_row, sqb)


def kernel(hidden_features, batch, current_epoch, sw1, sb1, sw2, sb2, sw3,
           sb3, fw1, fb1, fw2, fb2):
    h = jnp.concatenate(
        [hidden_features, hidden_features.mean(axis=1, keepdims=True)], axis=-1)
    x = jax.nn.relu(h @ sw1 + sb1)
    x = jax.nn.relu(x @ sw2 + sb2)
    s = jax.nn.relu(x @ sw3 + sb3)
    nrm = jnp.linalg.norm(s, axis=-1, keepdims=True)
    s = s / jnp.maximum(nrm, 1e-12)

    sq = jnp.sum(s * s, axis=1)
    s_pad = jnp.pad(s, ((0, NPAD - N), (0, 0)))
    sq_pad = jnp.pad(sq, (0, NPAD - N))

    idx, keys, w = _topk(s_pad, sq_pad)
    idx = idx[:N]
    keys = keys[:N].reshape(-1)
    dd = jnp.sum((s[idx] - s[:, None, :]) ** 2, axis=-1)
    w = jnp.exp(-1.0 * dd / (0.3 ** 2))

    keys_sorted = jnp.sort(keys)
    start = keys_sorted // N
    end = keys_sorted % N
    edge_index = jnp.stack([start, end])

    agg = jnp.sum(w[:, :, None] * h[idx], axis=1)

    cat = jnp.concatenate([agg, h], axis=-1)
    y = jax.nn.relu(cat @ fw1 + fb1)
    out = jax.nn.relu(y @ fw2 + fb2)
    return (out, edge_index, s, jnp.float32(1.0))


# speedup vs baseline: 5.7441x; 1.3448x over previous
import functools

import jax
import jax.numpy as jnp
import numpy as np
from jax import lax
from jax.experimental import pallas as pl
from jax.experimental.pallas import tpu as pltpu
from jax.experimental.pallas import tpu_sc as plsc

N = 10000
D_HID = 128
EMB = 8
K = 16
NPAD = 10240
BM = 128
GRID = NPAD // BM
INV_R2 = -1.0 / (0.3 ** 2)


def _topk_body(s_blk, sT, sq_row, sqb, idx_ref, key_ref, w_ref):
    i = pl.program_id(0)
    dot = jax.lax.dot_general(
        s_blk[...], sT[...], (((1,), (0,)), ((), ())),
        preferred_element_type=jnp.float32)
    dist = (sqb[...][:, 0:1] + sq_row[...][0:1, :]) - 2.0 * dot
    lane = jax.lax.broadcasted_iota(jnp.int32, (BM, NPAD), 1)
    dist = jnp.where(lane >= N, jnp.float32(np.inf), dist)
    rows = jax.lax.broadcasted_iota(jnp.int32, (BM, 1), 0) + i * BM
    big = jnp.int32(2 ** 30)
    for k in range(K):
        m = jnp.min(dist, axis=1, keepdims=True)
        amin = jnp.min(jnp.where(dist == m, lane, big), axis=1,
                       keepdims=True)
        idx_ref[:, k:k + 1] = amin
        key_ref[:, k:k + 1] = amin * N + rows
        w_ref[:, k:k + 1] = jnp.exp((-1.0 * m) / 0.09)
        dist = jnp.where(lane == amin, jnp.float32(np.inf), dist)


def _topk(s_pad, sq_pad):
    sT = s_pad.T
    sq_row = jnp.tile(sq_pad[None, :], (8, 1))
    sqb = jnp.tile(sq_pad[:, None], (1, 8))
    out_shapes = (
        jax.ShapeDtypeStruct((NPAD, K), jnp.int32),
        jax.ShapeDtypeStruct((NPAD, K), jnp.int32),
        jax.ShapeDtypeStruct((NPAD, K), jnp.float32),
    )
    blk = lambda i: (i, 0)
    return pl.pallas_call(
        _topk_body,
        grid=(GRID,),
        in_specs=[
            pl.BlockSpec((BM, EMB), blk),
            pl.BlockSpec((EMB, NPAD), lambda i: (0, 0)),
            pl.BlockSpec((8, NPAD), lambda i: (0, 0)),
            pl.BlockSpec((BM, 8), blk),
        ],
        out_specs=(
            pl.BlockSpec((BM, K), blk),
            pl.BlockSpec((BM, K), blk),
            pl.BlockSpec((BM, K), blk),
        ),
        out_shape=out_shapes,
    )(s_pad, sT, sq_row, sqb)


NW = 32
NB = 320
NCPAD = NW * NB
TILE = 8
NT = NB // TILE
NCOL = EMB + 1


def _sc_agg_kernel(hid_hbm, idx_hbm, sflat_hbm, sc0, sc1, sc2, sc3, sc4, sc5,
                   sc6, sc7, sc8, agg_hbm, aggm_hbm, idx_v, scs, rb0, rb1,
                   scol0, scol1, tilebuf, tbm, sem0, sem1, semo):
    scols = (sc0, sc1, sc2, sc3, sc4, sc5, sc6, sc7, sc8)
    wid = lax.axis_index("c") * 16 + lax.axis_index("s")
    n0 = wid * NB
    pltpu.sync_copy(idx_hbm.at[pl.ds(n0 * K, NB * K)], idx_v)
    pltpu.sync_copy(sflat_hbm.at[pl.ds(n0 * 16, NB * 16)], scs)

    def gather(t, rb, scol, sem):
        isl = idx_v.at[pl.ds(pl.multiple_of(t * (TILE * K), 128), TILE * K)]
        pltpu.async_copy(hid_hbm.at[isl], rb, sem)
        for c in range(NCOL):
            pltpu.async_copy(scols[c].at[isl], scol.at[c], sem)

    def drain(t, rb, scol, sem):
        isl = idx_v.at[pl.ds(0, TILE * K)]
        pltpu.make_async_copy(hid_hbm.at[isl], rb, sem).wait()
        for c in range(NCOL):
            pltpu.make_async_copy(scols[c].at[isl], scol.at[c], sem).wait()

    def compute(t, rb, scol):
        base8 = t * TILE
        for u in range(TILE):
            sv = scs[pl.ds(pl.multiple_of((base8 + u) * 16, 16), 16)]
            d = jnp.zeros((16,), jnp.float32)
            for c in range(EMB):
                g = scol[c, pl.ds(16 * u, 16)]
                diff = g - jnp.full((16,), sv[c], jnp.float32)
                d = d + diff * diff
            w = jnp.exp((-1.0 * d) / 0.09)
            acc = [jnp.zeros((16,), jnp.float32) for _ in range(8)]
            mv = scol[EMB, pl.ds(16 * u, 16)]
            am = jnp.float32(0.0)
            for r in range(K):
                wr = jnp.full((16,), w[r], jnp.float32)
                am = am + w[r] * mv[r]
                for j in range(8):
                    acc[j] = acc[j] + wr * rb[u * K + r, pl.ds(16 * j, 16)]
            for j in range(8):
                tilebuf[u, pl.ds(16 * j, 16)] = acc[j]
            tbm[u, :] = jnp.full((16,), am, jnp.float32)
        pltpu.async_copy(
            tilebuf,
            agg_hbm.at[pl.ds(pl.multiple_of(n0 + base8, TILE), TILE)], semo)
        pltpu.async_copy(
            tbm,
            aggm_hbm.at[pl.ds(pl.multiple_of(n0 + base8, TILE), TILE)], semo)
        pltpu.make_async_copy(
            tilebuf, agg_hbm.at[pl.ds(0, TILE)], semo).wait()
        pltpu.make_async_copy(
            tbm, aggm_hbm.at[pl.ds(0, TILE)], semo).wait()

    gather(0, rb0, scol0, sem0)

    def pair_body(m, carry):
        t_a = 2 * m
        t_b = 2 * m + 1
        gather(t_b, rb1, scol1, sem1)
        drain(t_a, rb0, scol0, sem0)
        compute(t_a, rb0, scol0)

        @pl.when(t_b + 1 < NT)
        def _():
            gather(t_b + 1, rb0, scol0, sem0)

        drain(t_b, rb1, scol1, sem1)
        compute(t_b, rb1, scol1)
        return carry

    lax.fori_loop(0, NT // 2, pair_body, 0)


def _sc_agg(hidden, idx_pad, s_pad48, mean_pad):
    mesh = plsc.VectorSubcoreMesh(core_axis_name="c", subcore_axis_name="s")
    sflat = jnp.pad(s_pad48, ((0, 0), (0, 16 - EMB))).reshape(-1)
    scols = [s_pad48[:, c] for c in range(EMB)] + [mean_pad]
    kfn = functools.partial(
        pl.kernel, mesh=mesh,
        out_type=(jax.ShapeDtypeStruct((NCPAD, 128), jnp.float32),
                  jax.ShapeDtypeStruct((NCPAD, 16), jnp.float32)),
        scratch_types=[
            pltpu.VMEM((NB * K,), jnp.int32),
            pltpu.VMEM((NB * 16,), jnp.float32),
            pltpu.VMEM((TILE * K, 128), jnp.float32),
            pltpu.VMEM((TILE * K, 128), jnp.float32),
            pltpu.VMEM((NCOL, TILE * K), jnp.float32),
            pltpu.VMEM((NCOL, TILE * K), jnp.float32),
            pltpu.VMEM((TILE, 128), jnp.float32),
            pltpu.VMEM((TILE, 16), jnp.float32),
            pltpu.SemaphoreType.DMA,
            pltpu.SemaphoreType.DMA,
            pltpu.SemaphoreType.DMA,
        ],
    )(_sc_agg_kernel)
    return kfn(hidden, idx_pad.reshape(-1), sflat, *scols)


def kernel(hidden_features, batch, current_epoch, sw1, sb1, sw2, sb2, sw3,
           sb3, fw1, fb1, fw2, fb2):
    h = jnp.concatenate(
        [hidden_features, hidden_features.mean(axis=1, keepdims=True)], axis=-1)
    x = jax.nn.relu(h @ sw1 + sb1)
    x = jax.nn.relu(x @ sw2 + sb2)
    s = jax.nn.relu(x @ sw3 + sb3)
    nrm = jnp.linalg.norm(s, axis=-1, keepdims=True)
    s = s / jnp.maximum(nrm, 1e-12)

    sq = jnp.sum(s * s, axis=1)
    s_pad = jnp.pad(s, ((0, NPAD - N), (0, 0)))
    sq_pad = jnp.pad(sq, (0, NPAD - N))

    idx, keys, w_unused = _topk(s_pad, sq_pad)
    idx = idx[:N]
    keys = keys[:N].reshape(-1)

    keys_sorted = jnp.sort(keys)
    start = keys_sorted // N
    end = keys_sorted % N
    edge_index = jnp.stack([start, end])

    mean_pad = jnp.pad(h[:, 128], (0, NCPAD - N))
    idx_pad = jnp.pad(idx, ((0, NCPAD - N), (0, 0)))
    s_pad48 = jnp.pad(s, ((0, NCPAD - N), (0, 0)))
    agg128, aggm = _sc_agg(hidden_features, idx_pad, s_pad48, mean_pad)
    agg = jnp.concatenate([agg128[:N], aggm[:N, :1]], axis=-1)

    cat = jnp.concatenate([agg, h], axis=-1)
    y = jax.nn.relu(cat @ fw1 + fb1)
    out = jax.nn.relu(y @ fw2 + fb2)
    return (out, edge_index, s, jnp.float32(1.0))


# speedup vs baseline: 6.4804x; 1.1282x over previous
import functools

import jax
import jax.numpy as jnp
import numpy as np
from jax import lax
from jax.experimental import pallas as pl
from jax.experimental.pallas import tpu as pltpu
from jax.experimental.pallas import tpu_sc as plsc

N = 10000
D_HID = 128
EMB = 8
K = 16
NPAD = 10240
BM = 128
GRID = NPAD // BM
INV_R2 = -1.0 / (0.3 ** 2)


def _topk_body(s_blk, sT, sq_row, sqb, idx_ref, key_ref, w_ref):
    i = pl.program_id(0)
    dot = jax.lax.dot_general(
        s_blk[...], sT[...], (((1,), (0,)), ((), ())),
        preferred_element_type=jnp.float32)
    dist = (sqb[...][:, 0:1] + sq_row[...][0:1, :]) - 2.0 * dot
    lanef = jax.lax.broadcasted_iota(jnp.int32, (BM, NPAD), 1).astype(
        jnp.float32)
    dist = jnp.where(lanef >= float(N), jnp.float32(np.inf), dist)
    rows = jax.lax.broadcasted_iota(jnp.int32, (BM, 1), 0) + i * BM
    bigf = jnp.float32(2.0 ** 30)
    for k in range(K):
        m = jnp.min(dist, axis=1, keepdims=True)
        aminf = jnp.min(jnp.where(dist == m, lanef, bigf), axis=1,
                        keepdims=True)
        amin = aminf.astype(jnp.int32)
        idx_ref[:, k:k + 1] = amin
        key_ref[:, k:k + 1] = amin * N + rows
        w_ref[:, k:k + 1] = jnp.exp((-1.0 * m) / 0.09)
        dist = jnp.where(lanef == aminf, jnp.float32(np.inf), dist)


def _topk(s_pad, sq_pad):
    sT = s_pad.T
    sq_row = jnp.tile(sq_pad[None, :], (8, 1))
    sqb = jnp.tile(sq_pad[:, None], (1, 8))
    out_shapes = (
        jax.ShapeDtypeStruct((NPAD, K), jnp.int32),
        jax.ShapeDtypeStruct((NPAD, K), jnp.int32),
        jax.ShapeDtypeStruct((NPAD, K), jnp.float32),
    )
    blk = lambda i: (i, 0)
    return pl.pallas_call(
        _topk_body,
        grid=(GRID,),
        in_specs=[
            pl.BlockSpec((BM, EMB), blk),
            pl.BlockSpec((EMB, NPAD), lambda i: (0, 0)),
            pl.BlockSpec((8, NPAD), lambda i: (0, 0)),
            pl.BlockSpec((BM, 8), blk),
        ],
        out_specs=(
            pl.BlockSpec((BM, K), blk),
            pl.BlockSpec((BM, K), blk),
            pl.BlockSpec((BM, K), blk),
        ),
        out_shape=out_shapes,
    )(s_pad, sT, sq_row, sqb)


NW = 32
NB = 320
NCPAD = NW * NB
TILE = 8
NT = NB // TILE
NCOL = EMB + 1


def _sc_agg_kernel(hid_hbm, idx_hbm, sflat_hbm, sc0, sc1, sc2, sc3, sc4, sc5,
                   sc6, sc7, sc8, agg_hbm, aggm_hbm, idx_v, scs, rb0, rb1,
                   scol0, scol1, tilebuf, tbm, sem0, sem1, semo):
    scols = (sc0, sc1, sc2, sc3, sc4, sc5, sc6, sc7, sc8)
    wid = lax.axis_index("c") * 16 + lax.axis_index("s")
    n0 = wid * NB
    pltpu.sync_copy(idx_hbm.at[pl.ds(n0 * K, NB * K)], idx_v)
    pltpu.sync_copy(sflat_hbm.at[pl.ds(n0 * 16, NB * 16)], scs)

    def gather(t, rb, scol, sem):
        isl = idx_v.at[pl.ds(pl.multiple_of(t * (TILE * K), 128), TILE * K)]
        pltpu.async_copy(hid_hbm.at[isl], rb, sem)
        for c in range(NCOL):
            pltpu.async_copy(scols[c].at[isl], scol.at[c], sem)

    def drain(t, rb, scol, sem):
        isl = idx_v.at[pl.ds(0, TILE * K)]
        pltpu.make_async_copy(hid_hbm.at[isl], rb, sem).wait()
        for c in range(NCOL):
            pltpu.make_async_copy(scols[c].at[isl], scol.at[c], sem).wait()

    def compute(t, rb, scol):
        base8 = t * TILE
        for u in range(TILE):
            sv = scs[pl.ds(pl.multiple_of((base8 + u) * 16, 16), 16)]
            d = jnp.zeros((16,), jnp.float32)
            for c in range(EMB):
                g = scol[c, pl.ds(16 * u, 16)]
                diff = g - jnp.full((16,), sv[c], jnp.float32)
                d = d + diff * diff
            w = jnp.exp((-1.0 * d) / 0.09)
            acc = [jnp.zeros((16,), jnp.float32) for _ in range(8)]
            mv = scol[EMB, pl.ds(16 * u, 16)]
            am = jnp.float32(0.0)
            for r in range(K):
                wr = jnp.full((16,), w[r], jnp.float32)
                am = am + w[r] * mv[r]
                for j in range(8):
                    acc[j] = acc[j] + wr * rb[u * K + r, pl.ds(16 * j, 16)]
            for j in range(8):
                tilebuf[u, pl.ds(16 * j, 16)] = acc[j]
            tbm[u, :] = jnp.full((16,), am, jnp.float32)
        pltpu.async_copy(
            tilebuf,
            agg_hbm.at[pl.ds(pl.multiple_of(n0 + base8, TILE), TILE)], semo)
        pltpu.async_copy(
            tbm,
            aggm_hbm.at[pl.ds(pl.multiple_of(n0 + base8, TILE), TILE)], semo)
        pltpu.make_async_copy(
            tilebuf, agg_hbm.at[pl.ds(0, TILE)], semo).wait()
        pltpu.make_async_copy(
            tbm, aggm_hbm.at[pl.ds(0, TILE)], semo).wait()

    gather(0, rb0, scol0, sem0)

    def pair_body(m, carry):
        t_a = 2 * m
        t_b = 2 * m + 1
        gather(t_b, rb1, scol1, sem1)
        drain(t_a, rb0, scol0, sem0)
        compute(t_a, rb0, scol0)

        @pl.when(t_b + 1 < NT)
        def _():
            gather(t_b + 1, rb0, scol0, sem0)

        drain(t_b, rb1, scol1, sem1)
        compute(t_b, rb1, scol1)
        return carry

    lax.fori_loop(0, NT // 2, pair_body, 0)


def _sc_agg(hidden, idx_pad, s_pad48, mean_pad):
    mesh = plsc.VectorSubcoreMesh(core_axis_name="c", subcore_axis_name="s")
    sflat = jnp.pad(s_pad48, ((0, 0), (0, 16 - EMB))).reshape(-1)
    scols = [s_pad48[:, c] for c in range(EMB)] + [mean_pad]
    kfn = functools.partial(
        pl.kernel, mesh=mesh,
        out_type=(jax.ShapeDtypeStruct((NCPAD, 128), jnp.float32),
                  jax.ShapeDtypeStruct((NCPAD, 16), jnp.float32)),
        scratch_types=[
            pltpu.VMEM((NB * K,), jnp.int32),
            pltpu.VMEM((NB * 16,), jnp.float32),
            pltpu.VMEM((TILE * K, 128), jnp.float32),
            pltpu.VMEM((TILE * K, 128), jnp.float32),
            pltpu.VMEM((NCOL, TILE * K), jnp.float32),
            pltpu.VMEM((NCOL, TILE * K), jnp.float32),
            pltpu.VMEM((TILE, 128), jnp.float32),
            pltpu.VMEM((TILE, 16), jnp.float32),
            pltpu.SemaphoreType.DMA,
            pltpu.SemaphoreType.DMA,
            pltpu.SemaphoreType.DMA,
        ],
    )(_sc_agg_kernel)
    return kfn(hidden, idx_pad.reshape(-1), sflat, *scols)


BR = 400


def _fmlp_body(cat_ref, fw1_ref, fb1_ref, fw2_ref, fb2_ref, out_ref):
    y = jax.nn.relu(
        jnp.dot(cat_ref[...], fw1_ref[...],
                preferred_element_type=jnp.float32) + fb1_ref[...][0:1, :])
    out_ref[...] = jax.nn.relu(
        jnp.dot(y, fw2_ref[...],
                preferred_element_type=jnp.float32) + fb2_ref[...][0:1, :])


def _fmlp(cat, fw1, fb1, fw2, fb2):
    fb1t = jnp.tile(fb1[None, :], (8, 1))
    fb2t = jnp.tile(fb2[None, :], (8, 1))
    return pl.pallas_call(
        _fmlp_body,
        grid=(N // BR,),
        in_specs=[
            pl.BlockSpec((BR, 258), lambda i: (i, 0)),
            pl.BlockSpec((258, 128), lambda i: (0, 0)),
            pl.BlockSpec((8, 128), lambda i: (0, 0)),
            pl.BlockSpec((128, 128), lambda i: (0, 0)),
            pl.BlockSpec((8, 128), lambda i: (0, 0)),
        ],
        out_specs=pl.BlockSpec((BR, 128), lambda i: (i, 0)),
        out_shape=jax.ShapeDtypeStruct((N, 128), jnp.float32),
    )(cat, fw1, fb1t, fw2, fb2t)


def kernel(hidden_features, batch, current_epoch, sw1, sb1, sw2, sb2, sw3,
           sb3, fw1, fb1, fw2, fb2):
    h = jnp.concatenate(
        [hidden_features, hidden_features.mean(axis=1, keepdims=True)], axis=-1)
    x = jax.nn.relu(h @ sw1 + sb1)
    x = jax.nn.relu(x @ sw2 + sb2)
    s = jax.nn.relu(x @ sw3 + sb3)
    nrm = jnp.linalg.norm(s, axis=-1, keepdims=True)
    s = s / jnp.maximum(nrm, 1e-12)

    sq = jnp.sum(s * s, axis=1)
    s_pad = jnp.pad(s, ((0, NPAD - N), (0, 0)))
    sq_pad = jnp.pad(sq, (0, NPAD - N))

    idx, keys, w_unused = _topk(s_pad, sq_pad)
    idx = idx[:N]
    keys = keys[:N].reshape(-1)

    keys_sorted = jnp.sort(keys)
    start = keys_sorted // N
    end = keys_sorted % N
    edge_index = jnp.stack([start, end])

    mean_pad = jnp.pad(h[:, 128], (0, NCPAD - N))
    idx_pad = jnp.pad(idx, ((0, NCPAD - N), (0, 0)))
    s_pad48 = jnp.pad(s, ((0, NCPAD - N), (0, 0)))
    agg128, aggm = _sc_agg(hidden_features, idx_pad, s_pad48, mean_pad)
    agg = jnp.concatenate([agg128[:N], aggm[:N, :1]], axis=-1)

    cat = jnp.concatenate([agg, h], axis=-1)
    out = _fmlp(cat, fw1, fb1, fw2, fb2)
    return (out, edge_index, s, jnp.float32(1.0))
